# baseline (speedup 1.0000x reference)
import jax
import jax.numpy as jnp
from jax.experimental import pallas as pl
from jax.experimental.pallas import tpu as pltpu

_NUM_CODES = 512
_CDIM = 64
_PIX_TILE = 1024
_BETA = 0.25


def _vq_body(x_ref, embT_ref, emb_ref, zq_ref, codes_ref, loss_ref):
    x = x_ref[0]
    embT = embT_ref[...]
    emb = emb_ref[...]

    xsq = jnp.sum(x * x, axis=0)
    esq = jnp.sum(emb * emb, axis=1)
    m = jax.lax.dot_general(
        x, embT, (((0,), (0,)), ((), ())),
        preferred_element_type=jnp.float32)
    dist = (xsq[:, None] + esq[None, :]) - 2.0 * m

    mn = jnp.min(dist, axis=1, keepdims=True)
    lane = jax.lax.broadcasted_iota(jnp.int32, dist.shape, 1)
    codes = jnp.min(jnp.where(dist == mn, lane, _NUM_CODES), axis=1)
    codes_ref[0, 0, :] = codes

    onehot = (lane == codes[:, None]).astype(jnp.float32)
    zqT = jax.lax.dot_general(
        emb, onehot, (((0,), (1,)), ((), ())),
        precision=jax.lax.Precision.HIGHEST,
        preferred_element_type=jnp.float32)

    zq_ref[0] = x + (zqT - x)
    loss_ref[0, 0, 0] = jnp.sum((zqT - x) ** 2)


def kernel(z_e, emb):
    B, C, H, W = z_e.shape
    HW = H * W
    z3 = z_e.reshape(B, C, HW)

    zq_st, codes3, lossp = pl.pallas_call(
        _vq_body,
        grid=(B,),
        in_specs=[
            pl.BlockSpec((1, C, HW), lambda m: (m, 0, 0)),
            pl.BlockSpec((C, _NUM_CODES), lambda m: (0, 0)),
            pl.BlockSpec((_NUM_CODES, C), lambda m: (0, 0)),
        ],
        out_specs=[
            pl.BlockSpec((1, C, HW), lambda m: (m, 0, 0)),
            pl.BlockSpec((1, 1, HW), lambda m: (m, 0, 0)),
            pl.BlockSpec((1, 1, 1), lambda m: (m, 0, 0),
                         memory_space=pltpu.SMEM),
        ],
        out_shape=[
            jax.ShapeDtypeStruct((B, C, HW), jnp.float32),
            jax.ShapeDtypeStruct((B, 1, HW), jnp.int32),
            jax.ShapeDtypeStruct((B, 1, 1), jnp.float32),
        ],
    )(z3, emb.T, emb)

    zq_st = zq_st.reshape(B, C, H, W)
    codes = codes3.reshape(B, H, W)
    vq_loss = (1.0 + _BETA) * jnp.sum(lossp) / (B * C * H * W)
    return zq_st, vq_loss, codes


# speedup vs baseline: 1.9622x; 1.9622x over previous
import jax
import jax.numpy as jnp
from jax.experimental import pallas as pl
from jax.experimental.pallas import tpu as pltpu

_NUM_CODES = 512
_BETA = 0.25


def _vq_body(x_ref, emb2_ref, embT_hi_ref, embT_lo_ref, emb_ref,
             zq_ref, codes_ref, loss_ref):
    x = x_ref[0]
    emb2 = emb2_ref[...]
    emb = emb_ref[...]

    xsq = jnp.sum(x * x, axis=0)
    esq = jnp.sum(emb * emb, axis=1)
    m2 = jax.lax.dot_general(
        emb2, x, (((1,), (0,)), ((), ())),
        preferred_element_type=jnp.float32)
    dist = (esq[:, None] + xsq[None, :]) - m2

    mn = jnp.min(dist, axis=0, keepdims=True)
    code_iota = jax.lax.broadcasted_iota(jnp.int32, dist.shape, 0)
    sel = jnp.where(dist == mn, code_iota, _NUM_CODES)
    codes = jnp.min(sel, axis=0)
    codes_ref[0, 0, :] = codes

    onehot = (code_iota == codes[None, :]).astype(jnp.bfloat16)
    zqT = (jax.lax.dot_general(
               embT_hi_ref[...], onehot, (((1,), (0,)), ((), ())),
               preferred_element_type=jnp.float32)
           + jax.lax.dot_general(
               embT_lo_ref[...], onehot, (((1,), (0,)), ((), ())),
               preferred_element_type=jnp.float32))

    zq_ref[0] = x + (zqT - x)
    loss_ref[0, 0, 0] = jnp.sum((zqT - x) ** 2)


def kernel(z_e, emb):
    B, C, H, W = z_e.shape
    HW = H * W
    z3 = z_e.reshape(B, C, HW)

    embT = emb.T
    embT_hi = embT.astype(jnp.bfloat16)
    embT_lo = (embT - embT_hi.astype(jnp.float32)).astype(jnp.bfloat16)

    zq_st, codes3, lossp = pl.pallas_call(
        _vq_body,
        grid=(B,),
        in_specs=[
            pl.BlockSpec((1, C, HW), lambda m: (m, 0, 0)),
            pl.BlockSpec((_NUM_CODES, C), lambda m: (0, 0)),
            pl.BlockSpec((C, _NUM_CODES), lambda m: (0, 0)),
            pl.BlockSpec((C, _NUM_CODES), lambda m: (0, 0)),
            pl.BlockSpec((_NUM_CODES, C), lambda m: (0, 0)),
        ],
        out_specs=[
            pl.BlockSpec((1, C, HW), lambda m: (m, 0, 0)),
            pl.BlockSpec((1, 1, HW), lambda m: (m, 0, 0)),
            pl.BlockSpec((1, 1, 1), lambda m: (m, 0, 0),
                         memory_space=pltpu.SMEM),
        ],
        out_shape=[
            jax.ShapeDtypeStruct((B, C, HW), jnp.float32),
            jax.ShapeDtypeStruct((B, 1, HW), jnp.int32),
            jax.ShapeDtypeStruct((B, 1, 1), jnp.float32),
        ],
    )(z3, emb * 2.0, embT_hi, embT_lo, emb)

    zq_st = zq_st.reshape(B, C, H, W)
    codes = codes3.reshape(B, H, W)
    vq_loss = (1.0 + _BETA) * jnp.sum(lossp) / (B * C * H * W)
    return zq_st, vq_loss, codes


# speedup vs baseline: 2.3366x; 1.1908x over previous
import jax
import jax.numpy as jnp
from jax.experimental import pallas as pl
from jax.experimental.pallas import tpu as pltpu

_NUM_CODES = 512
_BETA = 0.25


def _vq_body(x_ref, emb2_ref, embT_hi_ref, embT_lo_ref, emb_ref,
             zq_ref, codes_ref, loss_ref):
    nb = x_ref.shape[0]
    x = jnp.concatenate([x_ref[i] for i in range(nb)], axis=1) \
        if nb > 1 else x_ref[0]
    emb2 = emb2_ref[...]
    emb = emb_ref[...]

    xsq = jnp.sum(x * x, axis=0)
    esq = jnp.sum(emb * emb, axis=1)
    m2 = jax.lax.dot_general(
        emb2, x, (((1,), (0,)), ((), ())),
        preferred_element_type=jnp.float32)
    dist = (esq[:, None] + xsq[None, :]) - m2

    mn = jnp.min(dist, axis=0, keepdims=True)
    code_iota = jax.lax.broadcasted_iota(jnp.int32, dist.shape, 0)
    sel = jnp.where(dist == mn, code_iota, _NUM_CODES)
    codes = jnp.min(sel, axis=0)
    HW = codes.shape[0] // nb
    for i in range(nb):
        codes_ref[i, 0, :] = codes[i * HW:(i + 1) * HW]

    loss_ref[0, 0, 0] = jnp.sum(mn)

    onehot = (code_iota == codes[None, :]).astype(jnp.bfloat16)
    zqT = (jax.lax.dot_general(
               embT_hi_ref[...], onehot, (((1,), (0,)), ((), ())),
               preferred_element_type=jnp.float32)
           + jax.lax.dot_general(
               embT_lo_ref[...], onehot, (((1,), (0,)), ((), ())),
               preferred_element_type=jnp.float32))

    zq_st = x + (zqT - x)
    for i in range(nb):
        zq_ref[i] = zq_st[:, i * HW:(i + 1) * HW]


def kernel(z_e, emb):
    B, C, H, W = z_e.shape
    HW = H * W
    NB = 2
    z3 = z_e.reshape(B, C, HW)

    embT = emb.T
    embT_hi = embT.astype(jnp.bfloat16)
    embT_lo = (embT - embT_hi.astype(jnp.float32)).astype(jnp.bfloat16)

    zq_st, codes3, lossp = pl.pallas_call(
        _vq_body,
        grid=(B // NB,),
        in_specs=[
            pl.BlockSpec((NB, C, HW), lambda m: (m, 0, 0)),
            pl.BlockSpec((_NUM_CODES, C), lambda m: (0, 0)),
            pl.BlockSpec((C, _NUM_CODES), lambda m: (0, 0)),
            pl.BlockSpec((C, _NUM_CODES), lambda m: (0, 0)),
            pl.BlockSpec((_NUM_CODES, C), lambda m: (0, 0)),
        ],
        out_specs=[
            pl.BlockSpec((NB, C, HW), lambda m: (m, 0, 0)),
            pl.BlockSpec((NB, 1, HW), lambda m: (m, 0, 0)),
            pl.BlockSpec((1, 1, 1), lambda m: (m, 0, 0),
                         memory_space=pltpu.SMEM),
        ],
        out_shape=[
            jax.ShapeDtypeStruct((B, C, HW), jnp.float32),
            jax.ShapeDtypeStruct((B, 1, HW), jnp.int32),
            jax.ShapeDtypeStruct((B // NB, 1, 1), jnp.float32),
        ],
    )(z3, emb * 2.0, embT_hi, embT_lo, emb)

    zq_st = zq_st.reshape(B, C, H, W)
    codes = codes3.reshape(B, H, W)
    vq_loss = (1.0 + _BETA) * jnp.sum(lossp) / (B * C * H * W)
    return zq_st, vq_loss, codes
